# baseline (speedup 1.0000x reference)
import functools

import jax
import jax.numpy as jnp
from jax import lax
from jax.experimental import pallas as pl
from jax.experimental.pallas import tpu as pltpu
from jax.experimental.pallas import tpu_sc as plsc

_N = 100000
_E = 3200000
_C = 16
_K = 10
_ALPHA = 0.9

_NC = 2
_NS = 16
_NW = _NC * _NS

_GRP = 128
_SCH = 8
_SCS = 4
_ROWS_PW = 792
_ROWS_PD = 784
_NSUP = _ROWS_PD // _SCH
_NSUS = _ROWS_PW // _SCS
_E_PAD = _NW * _ROWS_PW * _GRP
_NPAD = 100352
_RPT = _NPAD // _NS
_RPW = _NPAD // _NW
_CCH = 784

_mesh = plsc.VectorSubcoreMesh(core_axis_name="c", subcore_axis_name="s")
_params = pltpu.CompilerParams(use_tc_tiling_on_sc=False)


@functools.partial(
    pl.kernel,
    out_type=jax.ShapeDtypeStruct((_NC, _NPAD, _C), jnp.float32),
    mesh=_mesh,
    scratch_types=[
        pltpu.VMEM_SHARED((_NPAD, _C), jnp.float32),
        pltpu.VMEM((_SCS, _GRP), jnp.int32),
        pltpu.VMEM((_SCS, _GRP), jnp.int32),
        pltpu.VMEM((_SCS, _GRP, _C), jnp.float32),
        pltpu.VMEM((_SCS, _GRP), jnp.int32),
        pltpu.VMEM((_SCS, _GRP), jnp.int32),
        pltpu.VMEM((_SCS, _GRP, _C), jnp.float32),
        pltpu.VMEM((_SCS, _GRP), jnp.int32),
        pltpu.VMEM((_SCS, _GRP), jnp.int32),
        pltpu.VMEM((_SCS, _GRP, _C), jnp.float32),
        pltpu.SemaphoreType.DMA,
        pltpu.SemaphoreType.DMA,
        pltpu.SemaphoreType.DMA,
        pltpu.SemaphoreType.DMA,
        pltpu.SemaphoreType.DMA,
        pltpu.SemaphoreType.DMA,
        pltpu.SemaphoreType.DMA,
        pltpu.SemaphoreType.DMA,
        pltpu.SemaphoreType.DMA,
    ],
    compiler_params=_params,
)
def _spmm(z_hbm, src_hbm, dst_hbm, zeros_hbm, out_hbm,
          acc, src_v0, dst_v0, rows_v0, src_v1, dst_v1, rows_v1,
          src_v2, dst_v2, rows_v2,
          isem0, isem1, isem2, gsem0, gsem1, gsem2, ssem0, ssem1, ssem2):
    cid = lax.axis_index("c")
    sid = lax.axis_index("s")
    wid = cid * _NS + sid
    pltpu.sync_copy(zeros_hbm.at[pl.ds(sid * _RPT, _RPT)],
                    acc.at[pl.ds(sid * _RPT, _RPT)])
    plsc.subcore_barrier()
    base = wid * _ROWS_PW

    bufs = ((src_v0, dst_v0, rows_v0), (src_v1, dst_v1, rows_v1),
            (src_v2, dst_v2, rows_v2))
    isems = (isem0, isem1, isem2)
    gsems = (gsem0, gsem1, gsem2)
    ssems = (ssem0, ssem1, ssem2)

    def stage_a(g, s, drain):
        sv, dv, rv = bufs[s]
        if drain:
            for j in range(_SCS):
                pltpu.make_async_copy(rv.at[j], acc.at[dv.at[j]],
                                      ssems[s]).wait()
        row0 = base + g * _SCS
        pltpu.async_copy(src_hbm.at[pl.ds(row0, _SCS)], sv, isems[s])
        pltpu.async_copy(dst_hbm.at[pl.ds(row0, _SCS)], dv, isems[s])

    def stage_b(g, s):
        sv, dv, rv = bufs[s]
        row0 = base + g * _SCS
        pltpu.make_async_copy(src_hbm.at[pl.ds(row0, _SCS)], sv,
                              isems[s]).wait()
        pltpu.make_async_copy(dst_hbm.at[pl.ds(row0, _SCS)], dv,
                              isems[s]).wait()
        for j in range(_SCS):
            pltpu.async_copy(z_hbm.at[sv.at[j]], rv.at[j], gsems[s])

    def stage_c(g, s):
        sv, dv, rv = bufs[s]
        for j in range(_SCS):
            pltpu.make_async_copy(z_hbm.at[sv.at[j]], rv.at[j],
                                  gsems[s]).wait()
        for j in range(_SCS):
            pltpu.async_copy(rv.at[j], acc.at[dv.at[j]], ssems[s],
                             add=True)

    stage_a(0, 0, False)
    stage_a(1, 1, False)
    stage_b(0, 0)
    stage_a(2, 2, False)
    stage_b(1, 1)
    stage_c(0, 0)

    @pl.loop(0, (_NSUS - 3) // 3)
    def _loop(t):
        g = 3 * t + 1
        for k in range(3):
            stage_a(g + k + 2, (k + 0) % 3, True)
            stage_b(g + k + 1, (k + 2) % 3)
            stage_c(g + k, (k + 1) % 3)

    stage_b(_NSUS - 1, (_NSUS - 1) % 3)
    stage_c(_NSUS - 2, (_NSUS - 2) % 3)
    stage_c(_NSUS - 1, (_NSUS - 1) % 3)
    for g in range(_NSUS - 3, _NSUS):
        sv, dv, rv = bufs[g % 3]
        for j in range(_SCS):
            pltpu.make_async_copy(rv.at[j], acc.at[dv.at[j]],
                                  ssems[g % 3]).wait()
    plsc.subcore_barrier()
    pltpu.sync_copy(acc.at[pl.ds(sid * _RPT, _RPT)],
                    out_hbm.at[cid, pl.ds(sid * _RPT, _RPT)])


@functools.partial(
    pl.kernel,
    out_type=(jax.ShapeDtypeStruct((_NPAD,), jnp.float32),
              jax.ShapeDtypeStruct((_NPAD,), jnp.float32)),
    mesh=_mesh,
    scratch_types=[
        pltpu.VMEM_SHARED((_NPAD,), jnp.float32),
        pltpu.VMEM((_SCH, _GRP), jnp.int32),
        pltpu.VMEM((_SCH, _GRP), jnp.int32),
        pltpu.VMEM((_GRP,), jnp.float32),
        pltpu.SemaphoreType.DMA,
        pltpu.SemaphoreType.DMA,
    ],
    compiler_params=_params,
)
def _deg(src_hbm, dst_hbm, zeros1_hbm, out0_hbm, out1_hbm,
         dacc, idx_v0, idx_v1, ones_v, isem, ssem):
    cid = lax.axis_index("c")
    sid = lax.axis_index("s")
    wid = cid * _NS + sid
    for t in range(_GRP // 16):
        ones_v[pl.ds(t * 16, 16)] = jnp.full((16,), 1.0, jnp.float32)
    pltpu.sync_copy(zeros1_hbm.at[pl.ds(sid * _RPT, _RPT)],
                    dacc.at[pl.ds(sid * _RPT, _RPT)])
    plsc.subcore_barrier()
    base = wid * _ROWS_PD

    ibufs = (idx_v0, idx_v1)

    def chunk(idx_hbm, row0, b, first):
        iv = ibufs[b]
        if not first:
            for j in range(_SCH):
                pltpu.make_async_copy(ones_v, dacc.at[iv.at[j]], ssem).wait()
        pltpu.async_copy(idx_hbm.at[pl.ds(row0, _SCH)], iv, isem)
        pltpu.make_async_copy(idx_hbm.at[pl.ds(row0, _SCH)], iv, isem).wait()
        for j in range(_SCH):
            pltpu.async_copy(ones_v, dacc.at[iv.at[j]], ssem, add=True)

    chunk(src_hbm, base, 0, True)
    chunk(src_hbm, base + _SCH, 1, True)

    @pl.loop(0, (_NSUP - 2) // 2)
    def _loop_s(t):
        row0 = base + (2 * t + 2) * _SCH
        chunk(src_hbm, row0, 0, False)
        chunk(src_hbm, row0 + _SCH, 1, False)

    @pl.loop(0, _NSUP // 2)
    def _loop_d(t):
        row0 = base + 2 * t * _SCH
        chunk(dst_hbm, row0, 0, False)
        chunk(dst_hbm, row0 + _SCH, 1, False)

    for b in range(2):
        for j in range(_SCH):
            pltpu.make_async_copy(ones_v, dacc.at[ibufs[b].at[j]], ssem).wait()
    plsc.subcore_barrier()

    @pl.when(cid == 0)
    def _():
        pltpu.sync_copy(dacc.at[pl.ds(sid * _RPT, _RPT)],
                        out0_hbm.at[pl.ds(sid * _RPT, _RPT)])

    @pl.when(cid == 1)
    def _():
        pltpu.sync_copy(dacc.at[pl.ds(sid * _RPT, _RPT)],
                        out1_hbm.at[pl.ds(sid * _RPT, _RPT)])


@functools.partial(
    pl.kernel,
    out_type=jax.ShapeDtypeStruct((_NPAD, _C), jnp.float32),
    mesh=_mesh,
    scratch_types=[
        pltpu.VMEM((2, _CCH, _C), jnp.float32),
        pltpu.VMEM((2, _CCH, _C), jnp.float32),
        pltpu.VMEM((2, _CCH, _C), jnp.float32),
        pltpu.VMEM((2, _CCH, _C), jnp.float32),
        pltpu.VMEM((2, _CCH, _C), jnp.float32),
        pltpu.SemaphoreType.DMA,
        pltpu.SemaphoreType.DMA,
    ],
    compiler_params=_params,
)
def _rescale(p_hbm, s_hbm, b_hbm, out_hbm, p0_v, p1_v, s_v, b_v, o_v,
             sem, osem):
    cid = lax.axis_index("c")
    sid = lax.axis_index("s")
    wid = cid * _NS + sid
    base = wid * _RPW
    nch = _RPW // _CCH

    def fetch(t, b):
        row0 = base + t * _CCH
        pltpu.async_copy(p_hbm.at[0, pl.ds(row0, _CCH)], p0_v.at[b], sem)
        pltpu.async_copy(p_hbm.at[1, pl.ds(row0, _CCH)], p1_v.at[b], sem)
        pltpu.async_copy(s_hbm.at[pl.ds(row0, _CCH)], s_v.at[b], sem)
        pltpu.async_copy(b_hbm.at[pl.ds(row0, _CCH)], b_v.at[b], sem)

    def work(t, b, first):
        row0 = base + t * _CCH
        pltpu.make_async_copy(p_hbm.at[0, pl.ds(row0, _CCH)], p0_v.at[b], sem).wait()
        pltpu.make_async_copy(p_hbm.at[1, pl.ds(row0, _CCH)], p1_v.at[b], sem).wait()
        pltpu.make_async_copy(s_hbm.at[pl.ds(row0, _CCH)], s_v.at[b], sem).wait()
        pltpu.make_async_copy(b_hbm.at[pl.ds(row0, _CCH)], b_v.at[b], sem).wait()
        if not first:
            pltpu.make_async_copy(o_v.at[b], out_hbm.at[pl.ds(row0, _CCH)],
                                  osem).wait()
        pb0, pb1, sb, bb, ob = (p0_v.at[b], p1_v.at[b], s_v.at[b],
                                b_v.at[b], o_v.at[b])

        @pl.loop(0, _CCH, unroll=8)
        def _rows(r):
            ob[r] = sb[r] * (pb0[r] + pb1[r]) + bb[r]

        pltpu.async_copy(ob, out_hbm.at[pl.ds(row0, _CCH)], osem)

    fetch(0, 0)
    fetch(1, 1)
    work(0, 0, True)
    fetch(2, 0)
    work(1, 1, True)
    fetch(3, 1)
    work(2, 0, False)
    work(3, 1, False)
    for b in range(2):
        row0 = base + (2 + b) * _CCH
        pltpu.make_async_copy(o_v.at[b], out_hbm.at[pl.ds(row0, _CCH)],
                              osem).wait()


def kernel(redisuals, edge_index):
    r = redisuals
    ei = edge_index.astype(jnp.int32)
    padv = jnp.full((_E_PAD - _E,), _N, jnp.int32)
    srcp = jnp.concatenate([ei[0], padv]).reshape(_E_PAD // _GRP, _GRP)
    dstp = jnp.concatenate([ei[1], padv]).reshape(_E_PAD // _GRP, _GRP)
    padd = jnp.full((_NW * _ROWS_PD * _GRP - _E,), _N, jnp.int32)
    srcd = jnp.concatenate([ei[0], padd]).reshape(-1, _GRP)
    dstd = jnp.concatenate([ei[1], padd]).reshape(-1, _GRP)
    zeros2 = jnp.zeros((_NPAD, _C), jnp.float32)
    zeros1 = jnp.zeros((_NPAD,), jnp.float32)

    deg0, deg1 = _deg(srcd, dstd, zeros1)
    deg = deg0[:_N] + deg1[:_N]
    dis = jnp.where(deg > 0, lax.rsqrt(deg), 0.0)
    dinv = dis * dis

    zpad = jnp.zeros((_NPAD - _N, _C), jnp.float32)
    z = jnp.concatenate([dis[:, None] * r, zpad])
    s_a = jnp.concatenate([jnp.broadcast_to((_ALPHA * dinv)[:, None], (_N, _C)), zpad])
    b_a = jnp.concatenate([((1.0 - _ALPHA) * dis)[:, None] * r, zpad])
    s_f = jnp.concatenate([jnp.broadcast_to((_ALPHA * dis)[:, None], (_N, _C)), zpad])
    b_f = jnp.concatenate([(1.0 - _ALPHA) * r, zpad])

    for _ in range(_K - 1):
        p = _spmm(z, srcp, dstp, zeros2)
        z = _rescale(p, s_a, b_a)
    p = _spmm(z, srcp, dstp, zeros2)
    return _rescale(p, s_f, b_f)[:_N]


# speedup vs baseline: 1.0585x; 1.0585x over previous
import functools

import jax
import jax.numpy as jnp
from jax import lax
from jax.experimental import pallas as pl
from jax.experimental.pallas import tpu as pltpu
from jax.experimental.pallas import tpu_sc as plsc

_N = 100000
_E = 3200000
_C = 16
_K = 10
_ALPHA = 0.9

_NC = 2
_NS = 16
_NW = _NC * _NS

_GRP = 128
_SCH = 8
_SCS = 4
_ROWS_PW = 784
_ROWS_PD = 784
_NSUP = _ROWS_PD // _SCH
_NSUS = _ROWS_PW // _SCS
_E_PAD = _NW * _ROWS_PW * _GRP
_NPAD = 100352
_RPT = _NPAD // _NS
_RPW = _NPAD // _NW
_CCH = 784

_mesh = plsc.VectorSubcoreMesh(core_axis_name="c", subcore_axis_name="s")
_params = pltpu.CompilerParams(use_tc_tiling_on_sc=False)


@functools.partial(
    pl.kernel,
    out_type=jax.ShapeDtypeStruct((_NC, _NPAD, _C), jnp.float32),
    mesh=_mesh,
    scratch_types=[
        pltpu.VMEM_SHARED((_NPAD, _C), jnp.float32),
        pltpu.VMEM((_SCS, _GRP), jnp.int32),
        pltpu.VMEM((_SCS, _GRP), jnp.int32),
        pltpu.VMEM((_SCS, _GRP, _C), jnp.float32),
        pltpu.VMEM((_SCS, _GRP), jnp.int32),
        pltpu.VMEM((_SCS, _GRP), jnp.int32),
        pltpu.VMEM((_SCS, _GRP, _C), jnp.float32),
        pltpu.SemaphoreType.DMA,
        pltpu.SemaphoreType.DMA,
        pltpu.SemaphoreType.DMA,
    ],
    compiler_params=_params,
)
def _spmm(z_hbm, src_hbm, dst_hbm, zeros_hbm, out_hbm,
          acc, src_v0, dst_v0, rows_v0, src_v1, dst_v1, rows_v1,
          isem, gsem, ssem):
    cid = lax.axis_index("c")
    sid = lax.axis_index("s")
    wid = cid * _NS + sid
    pltpu.sync_copy(zeros_hbm.at[pl.ds(sid * _RPT, _RPT)],
                    acc.at[pl.ds(sid * _RPT, _RPT)])
    plsc.subcore_barrier()
    base = wid * _ROWS_PW

    bufs = ((src_v0, dst_v0, rows_v0), (src_v1, dst_v1, rows_v1))

    def chunk(row0, b, first):
        sv, dv, rv = bufs[b]
        if not first:
            for j in range(_SCS):
                pltpu.make_async_copy(rv.at[j], acc.at[dv.at[j]], ssem).wait()
        pltpu.async_copy(src_hbm.at[pl.ds(row0, _SCS)], sv, isem)
        pltpu.async_copy(dst_hbm.at[pl.ds(row0, _SCS)], dv, isem)
        pltpu.make_async_copy(src_hbm.at[pl.ds(row0, _SCS)], sv, isem).wait()
        pltpu.make_async_copy(dst_hbm.at[pl.ds(row0, _SCS)], dv, isem).wait()
        for j in range(_SCS):
            pltpu.async_copy(z_hbm.at[sv.at[j]], rv.at[j], gsem)
        for j in range(_SCS):
            pltpu.make_async_copy(z_hbm.at[sv.at[j]], rv.at[j], gsem).wait()
        for j in range(_SCS):
            pltpu.async_copy(rv.at[j], acc.at[dv.at[j]], ssem, add=True)

    chunk(base, 0, True)
    chunk(base + _SCS, 1, True)

    @pl.loop(0, (_NSUS - 2) // 2)
    def _loop(t):
        row0 = base + (2 * t + 2) * _SCS
        chunk(row0, 0, False)
        chunk(row0 + _SCS, 1, False)

    for b in range(2):
        sv, dv, rv = bufs[b]
        for j in range(_SCS):
            pltpu.make_async_copy(rv.at[j], acc.at[dv.at[j]], ssem).wait()
    plsc.subcore_barrier()
    pltpu.sync_copy(acc.at[pl.ds(sid * _RPT, _RPT)],
                    out_hbm.at[cid, pl.ds(sid * _RPT, _RPT)])


@functools.partial(
    pl.kernel,
    out_type=(jax.ShapeDtypeStruct((_NPAD,), jnp.float32),
              jax.ShapeDtypeStruct((_NPAD,), jnp.float32)),
    mesh=_mesh,
    scratch_types=[
        pltpu.VMEM_SHARED((_NPAD,), jnp.float32),
        pltpu.VMEM((_SCH, _GRP), jnp.int32),
        pltpu.VMEM((_SCH, _GRP), jnp.int32),
        pltpu.VMEM((_GRP,), jnp.float32),
        pltpu.SemaphoreType.DMA,
        pltpu.SemaphoreType.DMA,
    ],
    compiler_params=_params,
)
def _deg(src_hbm, dst_hbm, zeros1_hbm, out0_hbm, out1_hbm,
         dacc, idx_v0, idx_v1, ones_v, isem, ssem):
    cid = lax.axis_index("c")
    sid = lax.axis_index("s")
    wid = cid * _NS + sid
    for t in range(_GRP // 16):
        ones_v[pl.ds(t * 16, 16)] = jnp.full((16,), 1.0, jnp.float32)
    pltpu.sync_copy(zeros1_hbm.at[pl.ds(sid * _RPT, _RPT)],
                    dacc.at[pl.ds(sid * _RPT, _RPT)])
    plsc.subcore_barrier()
    base = wid * _ROWS_PD

    ibufs = (idx_v0, idx_v1)

    def chunk(idx_hbm, row0, b, first):
        iv = ibufs[b]
        if not first:
            for j in range(_SCH):
                pltpu.make_async_copy(ones_v, dacc.at[iv.at[j]], ssem).wait()
        pltpu.async_copy(idx_hbm.at[pl.ds(row0, _SCH)], iv, isem)
        pltpu.make_async_copy(idx_hbm.at[pl.ds(row0, _SCH)], iv, isem).wait()
        for j in range(_SCH):
            pltpu.async_copy(ones_v, dacc.at[iv.at[j]], ssem, add=True)

    chunk(src_hbm, base, 0, True)
    chunk(src_hbm, base + _SCH, 1, True)

    @pl.loop(0, (_NSUP - 2) // 2)
    def _loop_s(t):
        row0 = base + (2 * t + 2) * _SCH
        chunk(src_hbm, row0, 0, False)
        chunk(src_hbm, row0 + _SCH, 1, False)

    @pl.loop(0, _NSUP // 2)
    def _loop_d(t):
        row0 = base + 2 * t * _SCH
        chunk(dst_hbm, row0, 0, False)
        chunk(dst_hbm, row0 + _SCH, 1, False)

    for b in range(2):
        for j in range(_SCH):
            pltpu.make_async_copy(ones_v, dacc.at[ibufs[b].at[j]], ssem).wait()
    plsc.subcore_barrier()

    @pl.when(cid == 0)
    def _():
        pltpu.sync_copy(dacc.at[pl.ds(sid * _RPT, _RPT)],
                        out0_hbm.at[pl.ds(sid * _RPT, _RPT)])

    @pl.when(cid == 1)
    def _():
        pltpu.sync_copy(dacc.at[pl.ds(sid * _RPT, _RPT)],
                        out1_hbm.at[pl.ds(sid * _RPT, _RPT)])


@functools.partial(
    pl.kernel,
    out_type=jax.ShapeDtypeStruct((_NPAD, _C), jnp.float32),
    mesh=_mesh,
    scratch_types=[
        pltpu.VMEM((2, _CCH, _C), jnp.float32),
        pltpu.VMEM((2, _CCH, _C), jnp.float32),
        pltpu.VMEM((2, _CCH, _C), jnp.float32),
        pltpu.VMEM((2, _CCH, _C), jnp.float32),
        pltpu.VMEM((2, _CCH, _C), jnp.float32),
        pltpu.SemaphoreType.DMA,
        pltpu.SemaphoreType.DMA,
    ],
    compiler_params=_params,
)
def _rescale(p_hbm, s_hbm, b_hbm, out_hbm, p0_v, p1_v, s_v, b_v, o_v,
             sem, osem):
    cid = lax.axis_index("c")
    sid = lax.axis_index("s")
    wid = cid * _NS + sid
    base = wid * _RPW
    nch = _RPW // _CCH

    def fetch(t, b):
        row0 = base + t * _CCH
        pltpu.async_copy(p_hbm.at[0, pl.ds(row0, _CCH)], p0_v.at[b], sem)
        pltpu.async_copy(p_hbm.at[1, pl.ds(row0, _CCH)], p1_v.at[b], sem)
        pltpu.async_copy(s_hbm.at[pl.ds(row0, _CCH)], s_v.at[b], sem)
        pltpu.async_copy(b_hbm.at[pl.ds(row0, _CCH)], b_v.at[b], sem)

    def work(t, b, first):
        row0 = base + t * _CCH
        pltpu.make_async_copy(p_hbm.at[0, pl.ds(row0, _CCH)], p0_v.at[b], sem).wait()
        pltpu.make_async_copy(p_hbm.at[1, pl.ds(row0, _CCH)], p1_v.at[b], sem).wait()
        pltpu.make_async_copy(s_hbm.at[pl.ds(row0, _CCH)], s_v.at[b], sem).wait()
        pltpu.make_async_copy(b_hbm.at[pl.ds(row0, _CCH)], b_v.at[b], sem).wait()
        if not first:
            pltpu.make_async_copy(o_v.at[b], out_hbm.at[pl.ds(row0, _CCH)],
                                  osem).wait()
        pb0, pb1, sb, bb, ob = (p0_v.at[b], p1_v.at[b], s_v.at[b],
                                b_v.at[b], o_v.at[b])

        @pl.loop(0, _CCH, unroll=8)
        def _rows(r):
            ob[r] = sb[r] * (pb0[r] + pb1[r]) + bb[r]

        pltpu.async_copy(ob, out_hbm.at[pl.ds(row0, _CCH)], osem)

    fetch(0, 0)
    fetch(1, 1)
    work(0, 0, True)
    fetch(2, 0)
    work(1, 1, True)
    fetch(3, 1)
    work(2, 0, False)
    work(3, 1, False)
    for b in range(2):
        row0 = base + (2 + b) * _CCH
        pltpu.make_async_copy(o_v.at[b], out_hbm.at[pl.ds(row0, _CCH)],
                              osem).wait()


def kernel(redisuals, edge_index):
    r = redisuals
    ei = edge_index.astype(jnp.int32)
    padv = jnp.full((_E_PAD - _E,), _N, jnp.int32)
    srcp = jnp.concatenate([ei[0], padv]).reshape(_E_PAD // _GRP, _GRP)
    dstp = jnp.concatenate([ei[1], padv]).reshape(_E_PAD // _GRP, _GRP)
    zeros2 = jnp.zeros((_NPAD, _C), jnp.float32)
    zeros1 = jnp.zeros((_NPAD,), jnp.float32)

    deg0, deg1 = _deg(srcp, dstp, zeros1)
    deg = deg0[:_N] + deg1[:_N]
    dis = jnp.where(deg > 0, lax.rsqrt(deg), 0.0)
    dinv = dis * dis

    zpad = jnp.zeros((_NPAD - _N, _C), jnp.float32)
    z = jnp.concatenate([dis[:, None] * r, zpad])
    s_a = jnp.concatenate([jnp.broadcast_to((_ALPHA * dinv)[:, None], (_N, _C)), zpad])
    b_a = jnp.concatenate([((1.0 - _ALPHA) * dis)[:, None] * r, zpad])
    s_f = jnp.concatenate([jnp.broadcast_to((_ALPHA * dis)[:, None], (_N, _C)), zpad])
    b_f = jnp.concatenate([(1.0 - _ALPHA) * r, zpad])

    for _ in range(_K - 1):
        p = _spmm(z, srcp, dstp, zeros2)
        z = _rescale(p, s_a, b_a)
    p = _spmm(z, srcp, dstp, zeros2)
    return _rescale(p, s_f, b_f)[:_N]
